# baseline (speedup 1.0000x reference)
import functools

import jax
import jax.numpy as jnp
from jax import lax
from jax.experimental import pallas as pl
from jax.experimental.pallas import tpu as pltpu
from jax.experimental.pallas import tpu_sc as plsc

_B = 4
_C = 8
_S = 8192
_N = 13
_NP = 16
_G = 48
_K = 6
_D = 8
_NROWS = _B * _S
_PK = 16
_PR = _NROWS // _PK
_NIDX = _B * _S * _NP
_NC, _NS = 2, 16
_NW = _NC * _NS
_IPW = _NIDX // _NW
_NCH = 2
_IPC = _IPW // _NCH


def _make_sc_gather():
    mesh = plsc.VectorSubcoreMesh(core_axis_name="c", subcore_axis_name="s")

    @functools.partial(
        pl.kernel,
        out_type=jax.ShapeDtypeStruct((_NIDX, _D), jnp.float32),
        mesh=mesh,
        scratch_types=[
            pltpu.VMEM((_IPC,), jnp.int32),
            pltpu.VMEM((_IPC, _D), jnp.float32),
            pltpu.SemaphoreType.DMA,
        ],
        compiler_params=pltpu.CompilerParams(use_tc_tiling_on_sc=False),
    )
    def gather_k(table_hbm, idx_hbm, out_hbm, idx_v, rows_v, sem):
        wid = lax.axis_index("s") * _NC + lax.axis_index("c")
        for t in range(_NCH):
            base = wid * _IPW + t * _IPC
            pltpu.sync_copy(idx_hbm.at[pl.ds(base, _IPC)], idx_v)
            pltpu.async_copy(table_hbm.at[idx_v], rows_v, sem).wait()
            pltpu.sync_copy(rows_v, out_hbm.at[pl.ds(base, _IPC)])

    return gather_k


def _bdot(a, b):
    return lax.dot_general(a, b, (((1,), (0,)), ((), ())),
                           preferred_element_type=jnp.float32)


def _layer_body(r_ref, w_ref, b_ref, m_ref, o_ref):
    r = r_ref[...]
    rhi = r.astype(jnp.bfloat16)
    rlo = (r - rhi.astype(jnp.float32)).astype(jnp.bfloat16)
    w = w_ref[...]
    whi = w.astype(jnp.bfloat16)
    wlo = (w - whi.astype(jnp.float32)).astype(jnp.bfloat16)
    x = _bdot(rhi, whi) + (_bdot(rhi, wlo) + _bdot(rlo, whi))
    x = x + b_ref[...]
    sp = jnp.maximum(x, 0.0) + jnp.log(1.0 + jnp.exp(-jnp.abs(x)))
    o_ref[...] = _bdot(sp.astype(jnp.bfloat16), m_ref[...]) * (1.0 / _G)


def _layer_tc(rows, w, bias, nout, bz=4096):
    go = w.shape[1]
    mavg = jnp.tile(jnp.eye(nout, _D), (_G, 1)).astype(jnp.bfloat16)
    return pl.pallas_call(
        _layer_body,
        grid=(_NROWS // bz,),
        in_specs=[
            pl.BlockSpec((bz, _NP * _D), lambda i: (i, 0)),
            pl.BlockSpec((_NP * _D, go), lambda i: (0, 0)),
            pl.BlockSpec((1, go), lambda i: (0, 0)),
            pl.BlockSpec((go, _D), lambda i: (0, 0)),
        ],
        out_specs=pl.BlockSpec((bz, _D), lambda i: (i, 0)),
        out_shape=jax.ShapeDtypeStruct((_NROWS, _D), jnp.float32),
    )(rows, w, bias, mavg)


def _final_body(r_ref, sh_ref, k1_ref, d1_ref, o_ref):
    b = pl.program_id(0)
    j = pl.program_id(1)
    sh = sh_ref[...]
    kio = lax.broadcasted_iota(jnp.int32, (1, _K * _G), 1) // _G
    mask2 = (sh == kio).astype(jnp.float32)
    c1 = jnp.dot(r_ref[...], k1_ref[...], preferred_element_type=jnp.float32,
                 precision=lax.Precision.HIGHEST)
    s1 = jnp.sum(mask2 * c1, axis=0, keepdims=True)
    part = jnp.dot(s1, d1_ref[...], preferred_element_type=jnp.float32,
                   precision=lax.Precision.HIGHEST)

    @pl.when((b == 0) & (j == 0))
    def _():
        o_ref[...] = jnp.zeros_like(o_ref)

    o_ref[pl.ds(b, 1), :] += part


def _final_tc(rows, shells2d, k1, d1, bz=2048):
    nsb = _S // bz
    kg = k1.shape[1]
    return pl.pallas_call(
        _final_body,
        grid=(_B, nsb),
        in_specs=[
            pl.BlockSpec((bz, _NP * _D), lambda b, j: (b * nsb + j, 0)),
            pl.BlockSpec((bz, 1), lambda b, j: (b * nsb + j, 0)),
            pl.BlockSpec((_NP * _D, kg), lambda b, j: (0, 0)),
            pl.BlockSpec((kg, 3), lambda b, j: (0, 0)),
        ],
        out_specs=pl.BlockSpec((_B, 3), lambda b, j: (0, 0)),
        out_shape=jax.ShapeDtypeStruct((_B, 3), jnp.float32),
    )(rows, shells2d, k1, d1)


def kernel(InState, NNsites, GnnPerms, SitesToShells, gdiags,
           Psi0, b0, Psi1, b1, Psi2, b2, Psi3, b3, Psi4, b4, PsiR, VR):
    f32 = jnp.float32
    x = jnp.transpose(InState, (0, 2, 1))
    x = jnp.pad(x, ((0, 0), (0, 0), (0, _D - x.shape[2])))
    table = x.reshape(_PR, _PK * _D)

    self_idx = jnp.broadcast_to(
        jnp.arange(_S, dtype=NNsites.dtype)[:, None], (_S, _NP - _N))
    nn = jnp.concatenate([NNsites.T, self_idx], axis=1)
    idx = (jnp.arange(_B, dtype=jnp.int32)[:, None, None] * _S
           + nn[None, :, :]).reshape(_NIDX)

    ws, bs, nos = [], [], []
    for psi, bias in ((Psi0, b0), (Psi1, b1), (Psi2, b2), (Psi3, b3), (Psi4, b4)):
        o, cin, _ = psi.shape
        psip = jnp.pad(psi, ((0, 0), (0, _C - cin), (0, 0)))
        psig = psip[:, :, GnnPerms]
        w = jnp.transpose(psig, (3, 1, 2, 0)).reshape(_N * _C, _G * o)
        w = jnp.pad(w, ((0, (_NP - _N) * _C), (0, 0)))
        ws.append(w.astype(f32))
        bs.append(jnp.tile(bias, _G)[None, :].astype(f32))
        nos.append(o)

    sc_gather = _make_sc_gather()
    table = table.reshape(_NROWS, _D)
    for l in range(5):
        rows = sc_gather(table, idx).reshape(_NROWS, _NP * _D)
        table = _layer_tc(rows, ws[l], bs[l], nos[l])

    rows = sc_gather(table, idx).reshape(_NROWS, _NP * _D)
    psirg = PsiR[:, GnnPerms]
    k1 = jnp.zeros((_N, _C, _K * _G), f32)
    k1 = k1.at[:, 0, :].set(jnp.transpose(psirg, (2, 0, 1)).reshape(_N, _K * _G))
    k1 = k1.reshape(_N * _C, _K * _G)
    k1 = jnp.pad(k1, ((0, (_NP - _N) * _C), (0, 0)))
    d1 = (jnp.einsum('kd,gde->kge', VR, gdiags) / _G).reshape(_K * _G, 3)
    shells2d = jnp.tile(SitesToShells.astype(jnp.int32), _B)[:, None]
    return _final_tc(rows, shells2d, k1, d1.astype(f32))


# speedup vs baseline: 1.0298x; 1.0298x over previous
import functools

import jax
import jax.numpy as jnp
from jax import lax
from jax.experimental import pallas as pl
from jax.experimental.pallas import tpu as pltpu
from jax.experimental.pallas import tpu_sc as plsc

_B = 4
_C = 8
_S = 8192
_N = 13
_NP = 16
_G = 48
_K = 6
_D = 8
_NROWS = _B * _S
_PK = 16
_PR = _NROWS // _PK
_NIDX = _B * _S * _NP
_NC, _NS = 2, 16
_NW = _NC * _NS
_IPW = _NIDX // _NW
_NCH = 4
_IPC = _IPW // _NCH


def _make_sc_gather():
    mesh = plsc.VectorSubcoreMesh(core_axis_name="c", subcore_axis_name="s")

    @functools.partial(
        pl.kernel,
        out_type=jax.ShapeDtypeStruct((_NIDX, _D), jnp.float32),
        mesh=mesh,
        scratch_types=[
            pltpu.VMEM((_IPC,), jnp.int32),
            pltpu.VMEM((_IPC, _D), jnp.float32),
            pltpu.VMEM((_IPC, _D), jnp.float32),
            pltpu.SemaphoreType.DMA,
            pltpu.SemaphoreType.DMA,
        ],
        compiler_params=pltpu.CompilerParams(use_tc_tiling_on_sc=False),
    )
    def gather_k(table_hbm, idx_hbm, out_hbm, idx_v, rows0, rows1, semg, semw):
        wid = lax.axis_index("s") * _NC + lax.axis_index("c")
        rows = (rows0, rows1)
        for t in range(_NCH):
            base = wid * _IPW + t * _IPC
            pltpu.sync_copy(idx_hbm.at[pl.ds(base, _IPC)], idx_v)
            if t >= 2:
                pltpu.make_async_copy(
                    rows[t % 2],
                    out_hbm.at[pl.ds(base - 2 * _IPC, _IPC)], semw).wait()
            pltpu.async_copy(table_hbm.at[idx_v], rows[t % 2], semg).wait()
            pltpu.async_copy(rows[t % 2],
                             out_hbm.at[pl.ds(base, _IPC)], semw)
        for t in range(_NCH - 2, _NCH):
            base = wid * _IPW + t * _IPC
            pltpu.make_async_copy(
                rows[t % 2], out_hbm.at[pl.ds(base, _IPC)], semw).wait()

    return gather_k


def _bdot(a, b):
    return lax.dot_general(a, b, (((1,), (0,)), ((), ())),
                           preferred_element_type=jnp.float32)


def _layer_body(r_ref, w_ref, b_ref, m_ref, o_ref):
    r = r_ref[...]
    rhi = r.astype(jnp.bfloat16)
    rlo = (r - rhi.astype(jnp.float32)).astype(jnp.bfloat16)
    w = w_ref[...]
    whi = w.astype(jnp.bfloat16)
    wlo = (w - whi.astype(jnp.float32)).astype(jnp.bfloat16)
    x = _bdot(rhi, whi) + (_bdot(rhi, wlo) + _bdot(rlo, whi))
    x = x + b_ref[...]
    sp = jnp.maximum(x, 0.0) + jnp.log(1.0 + jnp.exp(-jnp.abs(x)))
    o_ref[...] = _bdot(sp.astype(jnp.bfloat16), m_ref[...]) * (1.0 / _G)


def _layer_tc(rows, w, bias, nout, bz=4096):
    go = w.shape[1]
    mavg = jnp.tile(jnp.eye(nout, _D), (_G, 1)).astype(jnp.bfloat16)
    return pl.pallas_call(
        _layer_body,
        grid=(_NROWS // bz,),
        in_specs=[
            pl.BlockSpec((bz, _NP * _D), lambda i: (i, 0)),
            pl.BlockSpec((_NP * _D, go), lambda i: (0, 0)),
            pl.BlockSpec((1, go), lambda i: (0, 0)),
            pl.BlockSpec((go, _D), lambda i: (0, 0)),
        ],
        out_specs=pl.BlockSpec((bz, _D), lambda i: (i, 0)),
        out_shape=jax.ShapeDtypeStruct((_NROWS, _D), jnp.float32),
    )(rows, w, bias, mavg)


def _final_body(r_ref, sh_ref, k1_ref, d1_ref, o_ref):
    b = pl.program_id(0)
    j = pl.program_id(1)
    sh = sh_ref[...]
    kio = lax.broadcasted_iota(jnp.int32, (1, _K * _G), 1) // _G
    mask2 = (sh == kio).astype(jnp.float32)
    r = r_ref[...]
    rhi = r.astype(jnp.bfloat16)
    rlo = (r - rhi.astype(jnp.float32)).astype(jnp.bfloat16)
    k1 = k1_ref[...]
    khi = k1.astype(jnp.bfloat16)
    klo = (k1 - khi.astype(jnp.float32)).astype(jnp.bfloat16)
    c1 = _bdot(rhi, khi) + (_bdot(rhi, klo) + _bdot(rlo, khi))
    s1 = jnp.sum(mask2 * c1, axis=0, keepdims=True)
    part = jnp.dot(s1, d1_ref[...], preferred_element_type=jnp.float32,
                   precision=lax.Precision.HIGHEST)

    @pl.when((b == 0) & (j == 0))
    def _():
        o_ref[...] = jnp.zeros_like(o_ref)

    o_ref[pl.ds(b, 1), :] += part


def _final_tc(rows, shells2d, k1, d1, bz=2048):
    nsb = _S // bz
    kg = k1.shape[1]
    return pl.pallas_call(
        _final_body,
        grid=(_B, nsb),
        in_specs=[
            pl.BlockSpec((bz, _NP * _D), lambda b, j: (b * nsb + j, 0)),
            pl.BlockSpec((bz, 1), lambda b, j: (b * nsb + j, 0)),
            pl.BlockSpec((_NP * _D, kg), lambda b, j: (0, 0)),
            pl.BlockSpec((kg, 3), lambda b, j: (0, 0)),
        ],
        out_specs=pl.BlockSpec((_B, 3), lambda b, j: (0, 0)),
        out_shape=jax.ShapeDtypeStruct((_B, 3), jnp.float32),
    )(rows, shells2d, k1, d1)


def kernel(InState, NNsites, GnnPerms, SitesToShells, gdiags,
           Psi0, b0, Psi1, b1, Psi2, b2, Psi3, b3, Psi4, b4, PsiR, VR):
    f32 = jnp.float32
    x = jnp.transpose(InState, (0, 2, 1))
    x = jnp.pad(x, ((0, 0), (0, 0), (0, _D - x.shape[2])))
    table = x.reshape(_PR, _PK * _D)

    self_idx = jnp.broadcast_to(
        jnp.arange(_S, dtype=NNsites.dtype)[:, None], (_S, _NP - _N))
    nn = jnp.concatenate([NNsites.T, self_idx], axis=1)
    idx = (jnp.arange(_B, dtype=jnp.int32)[:, None, None] * _S
           + nn[None, :, :]).reshape(_NIDX)

    ws, bs, nos = [], [], []
    for psi, bias in ((Psi0, b0), (Psi1, b1), (Psi2, b2), (Psi3, b3), (Psi4, b4)):
        o, cin, _ = psi.shape
        psip = jnp.pad(psi, ((0, 0), (0, _C - cin), (0, 0)))
        psig = psip[:, :, GnnPerms]
        w = jnp.transpose(psig, (3, 1, 2, 0)).reshape(_N * _C, _G * o)
        w = jnp.pad(w, ((0, (_NP - _N) * _C), (0, 0)))
        ws.append(w.astype(f32))
        bs.append(jnp.tile(bias, _G)[None, :].astype(f32))
        nos.append(o)

    sc_gather = _make_sc_gather()
    table = table.reshape(_NROWS, _D)
    for l in range(5):
        rows = sc_gather(table, idx).reshape(_NROWS, _NP * _D)
        table = _layer_tc(rows, ws[l], bs[l], nos[l])

    rows = sc_gather(table, idx).reshape(_NROWS, _NP * _D)
    psirg = PsiR[:, GnnPerms]
    k1 = jnp.zeros((_N, _C, _K * _G), f32)
    k1 = k1.at[:, 0, :].set(jnp.transpose(psirg, (2, 0, 1)).reshape(_N, _K * _G))
    k1 = k1.reshape(_N * _C, _K * _G)
    k1 = jnp.pad(k1, ((0, (_NP - _N) * _C), (0, 0)))
    d1 = (jnp.einsum('kd,gde->kge', VR, gdiags) / _G).reshape(_K * _G, 3)
    shells2d = jnp.tile(SitesToShells.astype(jnp.int32), _B)[:, None]
    return _final_tc(rows, shells2d, k1, d1.astype(f32))


# speedup vs baseline: 1.0533x; 1.0228x over previous
import functools

import jax
import jax.numpy as jnp
from jax import lax
from jax.experimental import pallas as pl
from jax.experimental.pallas import tpu as pltpu
from jax.experimental.pallas import tpu_sc as plsc

_B = 4
_C = 8
_S = 8192
_N = 13
_NP = 16
_G = 48
_K = 6
_D = 8
_NROWS = _B * _S
_PK = 16
_PR = _NROWS // _PK
_NIDX = _B * _S * _NP
_NC, _NS = 2, 16
_NW = _NC * _NS
_IPW = _NIDX // _NW
_NCH = 4
_IPC = _IPW // _NCH


def _make_sc_gather():
    mesh = plsc.VectorSubcoreMesh(core_axis_name="c", subcore_axis_name="s")

    @functools.partial(
        pl.kernel,
        out_type=jax.ShapeDtypeStruct((_NIDX, _D), jnp.float32),
        mesh=mesh,
        scratch_types=[
            pltpu.VMEM((_IPC,), jnp.int32),
            pltpu.VMEM((_IPC, _D), jnp.float32),
            pltpu.VMEM((_IPC, _D), jnp.float32),
            pltpu.SemaphoreType.DMA,
            pltpu.SemaphoreType.DMA,
        ],
        compiler_params=pltpu.CompilerParams(use_tc_tiling_on_sc=False),
    )
    def gather_k(table_hbm, idx_hbm, out_hbm, idx_v, rows0, rows1, semg, semw):
        wid = lax.axis_index("s") * _NC + lax.axis_index("c")
        rows = (rows0, rows1)
        for t in range(_NCH):
            base = wid * _IPW + t * _IPC
            pltpu.sync_copy(idx_hbm.at[pl.ds(base, _IPC)], idx_v)
            if t >= 2:
                pltpu.make_async_copy(
                    rows[t % 2],
                    out_hbm.at[pl.ds(base - 2 * _IPC, _IPC)], semw).wait()
            pltpu.async_copy(table_hbm.at[idx_v], rows[t % 2], semg).wait()
            pltpu.async_copy(rows[t % 2],
                             out_hbm.at[pl.ds(base, _IPC)], semw)
        for t in range(_NCH - 2, _NCH):
            base = wid * _IPW + t * _IPC
            pltpu.make_async_copy(
                rows[t % 2], out_hbm.at[pl.ds(base, _IPC)], semw).wait()

    return gather_k


def _bdot(a, b):
    return lax.dot_general(a, b, (((1,), (0,)), ((), ())),
                           preferred_element_type=jnp.float32)


def _layer_body(nout, r_ref, w_ref, b_ref, o_ref):
    j = pl.program_id(0)
    r = r_ref[...]
    rhi = r.astype(jnp.bfloat16)
    rlo = (r - rhi.astype(jnp.float32)).astype(jnp.bfloat16)
    w = w_ref[...]
    whi = w.astype(jnp.bfloat16)
    wlo = (w - whi.astype(jnp.float32)).astype(jnp.bfloat16)
    x = _bdot(rhi, whi) + (_bdot(rhi, wlo) + _bdot(rlo, whi))
    x = x + b_ref[...]
    sp = jnp.maximum(x, 0.0) + jnp.log(1.0 + jnp.exp(-jnp.abs(x)))
    go = w_ref.shape[1]
    rowo = lax.broadcasted_iota(jnp.int32, (go, 1), 0) % nout
    colio = lax.broadcasted_iota(jnp.int32, (1, _PK * _D), 1)
    m01 = jnp.where(colio == j * _D + rowo, 1.0, 0.0).astype(jnp.bfloat16)
    part = _bdot(sp.astype(jnp.bfloat16), m01) * (1.0 / _G)

    @pl.when(j == 0)
    def _():
        o_ref[...] = jnp.zeros_like(o_ref)

    o_ref[...] += part


def _layer_tc(rows, w, bias, nout):
    go = w.shape[1]
    return pl.pallas_call(
        functools.partial(_layer_body, nout),
        grid=(_PK,),
        in_specs=[
            pl.BlockSpec((_PR, _NP * _D), lambda j: (j, 0)),
            pl.BlockSpec((_NP * _D, go), lambda j: (0, 0)),
            pl.BlockSpec((1, go), lambda j: (0, 0)),
        ],
        out_specs=pl.BlockSpec((_PR, _PK * _D), lambda j: (0, 0)),
        out_shape=jax.ShapeDtypeStruct((_PR, _PK * _D), jnp.float32),
    )(rows, w, bias)


def _final_body(r_ref, sh_ref, k1_ref, d1_ref, o_ref):
    j = pl.program_id(0)
    sh = sh_ref[...]
    kio = lax.broadcasted_iota(jnp.int32, (1, _K * _G), 1) // _G
    mask2 = (sh == kio).astype(jnp.float32)
    r = r_ref[...]
    rhi = r.astype(jnp.bfloat16)
    rlo = (r - rhi.astype(jnp.float32)).astype(jnp.bfloat16)
    k1 = k1_ref[...]
    khi = k1.astype(jnp.bfloat16)
    klo = (k1 - khi.astype(jnp.float32)).astype(jnp.bfloat16)
    c1 = _bdot(rhi, khi) + (_bdot(rhi, klo) + _bdot(rlo, khi))
    e1 = (mask2 * c1).reshape(_B, _PR // _B, _K * _G)
    s1 = jnp.sum(e1, axis=1)
    part = jnp.dot(s1, d1_ref[...], preferred_element_type=jnp.float32,
                   precision=lax.Precision.HIGHEST)

    @pl.when(j == 0)
    def _():
        o_ref[...] = jnp.zeros_like(o_ref)

    o_ref[...] += part


def _final_tc(rows, shells2d, k1, d1):
    kg = k1.shape[1]
    return pl.pallas_call(
        _final_body,
        grid=(_PK,),
        in_specs=[
            pl.BlockSpec((_PR, _NP * _D), lambda j: (j, 0)),
            pl.BlockSpec((_PR, 1), lambda j: (j, 0)),
            pl.BlockSpec((_NP * _D, kg), lambda j: (0, 0)),
            pl.BlockSpec((kg, 3), lambda j: (0, 0)),
        ],
        out_specs=pl.BlockSpec((_B, 3), lambda j: (0, 0)),
        out_shape=jax.ShapeDtypeStruct((_B, 3), jnp.float32),
    )(rows, shells2d, k1, d1)


def kernel(InState, NNsites, GnnPerms, SitesToShells, gdiags,
           Psi0, b0, Psi1, b1, Psi2, b2, Psi3, b3, Psi4, b4, PsiR, VR):
    f32 = jnp.float32
    x = jnp.transpose(InState, (0, 2, 1))
    x = jnp.pad(x, ((0, 0), (0, 0), (0, _D - x.shape[2])))
    table = x.reshape(_PR, _PK * _D)

    self_idx = jnp.broadcast_to(
        jnp.arange(_S, dtype=NNsites.dtype)[:, None], (_S, _NP - _N))
    nn = jnp.concatenate([NNsites.T, self_idx], axis=1)
    idx = (jnp.arange(_B, dtype=jnp.int32)[:, None, None] * _S
           + nn[None, :, :]).reshape(_NROWS, _NP)
    idx = idx.reshape(_PR, _PK, _NP).transpose(1, 0, 2).reshape(_NIDX)

    ws, bs, nos = [], [], []
    for psi, bias in ((Psi0, b0), (Psi1, b1), (Psi2, b2), (Psi3, b3), (Psi4, b4)):
        o, cin, _ = psi.shape
        psip = jnp.pad(psi, ((0, 0), (0, _C - cin), (0, 0)))
        psig = psip[:, :, GnnPerms]
        w = jnp.transpose(psig, (3, 1, 2, 0)).reshape(_N * _C, _G * o)
        w = jnp.pad(w, ((0, (_NP - _N) * _C), (0, 0)))
        ws.append(w.astype(f32))
        bs.append(jnp.tile(bias, _G)[None, :].astype(f32))
        nos.append(o)

    sc_gather = _make_sc_gather()
    for l in range(5):
        rows = sc_gather(table.reshape(_NROWS, _D), idx).reshape(_NROWS, _NP * _D)
        table = _layer_tc(rows, ws[l], bs[l], nos[l])

    rows = sc_gather(table.reshape(_NROWS, _D), idx).reshape(_NROWS, _NP * _D)
    psirg = PsiR[:, GnnPerms]
    k1 = jnp.zeros((_N, _C, _K * _G), f32)
    k1 = k1.at[:, 0, :].set(jnp.transpose(psirg, (2, 0, 1)).reshape(_N, _K * _G))
    k1 = k1.reshape(_N * _C, _K * _G)
    k1 = jnp.pad(k1, ((0, (_NP - _N) * _C), (0, 0)))
    d1 = (jnp.einsum('kd,gde->kge', VR, gdiags) / _G).reshape(_K * _G, 3)
    shells2d = (jnp.tile(SitesToShells.astype(jnp.int32), _B)
                .reshape(_PR, _PK).T.reshape(_NROWS, 1))
    return _final_tc(rows, shells2d, k1, d1.astype(f32))


# speedup vs baseline: 1.1503x; 1.0921x over previous
import functools

import jax
import jax.numpy as jnp
from jax import lax
from jax.experimental import pallas as pl
from jax.experimental.pallas import tpu as pltpu
from jax.experimental.pallas import tpu_sc as plsc

_B = 4
_C = 8
_S = 8192
_N = 13
_NP = 16
_G = 48
_K = 6
_D = 8
_NROWS = _B * _S
_PK = 16
_PR = _NROWS // _PK
_NIDX = _B * _S * _NP
_NC, _NS = 2, 16
_NW = _NC * _NS
_IPW = _NIDX // _NW
_NCH = 4
_IPC = _IPW // _NCH


def _make_sc_gather():
    mesh = plsc.VectorSubcoreMesh(core_axis_name="c", subcore_axis_name="s")

    @functools.partial(
        pl.kernel,
        out_type=jax.ShapeDtypeStruct((_NIDX, _D), jnp.float32),
        mesh=mesh,
        scratch_types=[
            pltpu.VMEM((_IPC,), jnp.int32),
            pltpu.VMEM((_IPC, _D), jnp.float32),
            pltpu.VMEM((_IPC, _D), jnp.float32),
            pltpu.SemaphoreType.DMA,
            pltpu.SemaphoreType.DMA,
        ],
        compiler_params=pltpu.CompilerParams(use_tc_tiling_on_sc=False),
    )
    def gather_k(table_hbm, idx_hbm, out_hbm, idx_v, rows0, rows1, semg, semw):
        wid = lax.axis_index("s") * _NC + lax.axis_index("c")
        rows = (rows0, rows1)
        for t in range(_NCH):
            base = wid * _IPW + t * _IPC
            pltpu.sync_copy(idx_hbm.at[pl.ds(base, _IPC)], idx_v)
            if t >= 2:
                pltpu.make_async_copy(
                    rows[t % 2],
                    out_hbm.at[pl.ds(base - 2 * _IPC, _IPC)], semw).wait()
            pltpu.async_copy(table_hbm.at[idx_v], rows[t % 2], semg).wait()
            pltpu.async_copy(rows[t % 2],
                             out_hbm.at[pl.ds(base, _IPC)], semw)
        for t in range(_NCH - 2, _NCH):
            base = wid * _IPW + t * _IPC
            pltpu.make_async_copy(
                rows[t % 2], out_hbm.at[pl.ds(base, _IPC)], semw).wait()

    return gather_k


def _bdot(a, b):
    return lax.dot_general(a, b, (((1,), (0,)), ((), ())),
                           preferred_element_type=jnp.float32)


def _layer_body(nout, r_ref, w_ref, b_ref, o_ref):
    j = pl.program_id(0)
    r = r_ref[...]
    rhi = r.astype(jnp.bfloat16)
    rlo = (r - rhi.astype(jnp.float32)).astype(jnp.bfloat16)
    w = w_ref[...]
    whi = w.astype(jnp.bfloat16)
    wlo = (w - whi.astype(jnp.float32)).astype(jnp.bfloat16)
    x = _bdot(rhi, whi) + (_bdot(rhi, wlo) + _bdot(rlo, whi))
    x = x + b_ref[...]
    sp = jnp.maximum(x, 0.0) + jnp.log(1.0 + jnp.exp(-jnp.abs(x)))
    go = w_ref.shape[1]
    rowo = lax.broadcasted_iota(jnp.int32, (go, 1), 0) % nout
    colio = lax.broadcasted_iota(jnp.int32, (1, _PK * _D), 1)
    m01 = jnp.where(colio == j * _D + rowo, 1.0, 0.0).astype(jnp.bfloat16)
    part = _bdot(sp.astype(jnp.bfloat16), m01) * (1.0 / _G)

    @pl.when(j == 0)
    def _():
        o_ref[...] = jnp.zeros_like(o_ref)

    o_ref[...] += part


def _layer_tc(rows, w, bias, nout):
    go = w.shape[1]
    return pl.pallas_call(
        functools.partial(_layer_body, nout),
        grid=(_PK,),
        in_specs=[
            pl.BlockSpec((_PR, _NP * _D), lambda j: (j, 0)),
            pl.BlockSpec((_NP * _D, go), lambda j: (0, 0)),
            pl.BlockSpec((1, go), lambda j: (0, 0)),
        ],
        out_specs=pl.BlockSpec((_PR, _PK * _D), lambda j: (0, 0)),
        out_shape=jax.ShapeDtypeStruct((_PR, _PK * _D), jnp.float32),
    )(rows, w, bias)


def _final_body(r_ref, sh_ref, k1_ref, d1_ref, o_ref):
    j = pl.program_id(0)
    sh = sh_ref[...]
    kio = lax.broadcasted_iota(jnp.int32, (1, _K * _G), 1) // _G
    mask2 = (sh == kio).astype(jnp.float32)
    r = r_ref[...]
    rhi = r.astype(jnp.bfloat16)
    rlo = (r - rhi.astype(jnp.float32)).astype(jnp.bfloat16)
    k1 = k1_ref[...]
    khi = k1.astype(jnp.bfloat16)
    klo = (k1 - khi.astype(jnp.float32)).astype(jnp.bfloat16)
    c1 = _bdot(rhi, khi) + (_bdot(rhi, klo) + _bdot(rlo, khi))
    s1 = jnp.sum(mask2 * c1, axis=0, keepdims=True)
    part = jnp.dot(s1, d1_ref[...], preferred_element_type=jnp.float32,
                   precision=lax.Precision.HIGHEST)

    @pl.when(j == 0)
    def _():
        o_ref[...] = jnp.zeros_like(o_ref)

    o_ref[pl.ds(j // (_PK // _B), 1), :] += part


def _final_tc(rows, shells2d, k1, d1):
    kg = k1.shape[1]
    return pl.pallas_call(
        _final_body,
        grid=(_PK,),
        in_specs=[
            pl.BlockSpec((_PR, _NP * _D), lambda j: (j, 0)),
            pl.BlockSpec((_PR, 1), lambda j: (j, 0)),
            pl.BlockSpec((_NP * _D, kg), lambda j: (0, 0)),
            pl.BlockSpec((kg, 3), lambda j: (0, 0)),
        ],
        out_specs=pl.BlockSpec((_B, 3), lambda j: (0, 0)),
        out_shape=jax.ShapeDtypeStruct((_B, 3), jnp.float32),
    )(rows, shells2d, k1, d1)


def kernel(InState, NNsites, GnnPerms, SitesToShells, gdiags,
           Psi0, b0, Psi1, b1, Psi2, b2, Psi3, b3, Psi4, b4, PsiR, VR):
    f32 = jnp.float32
    x = jnp.transpose(InState, (0, 2, 1))
    x = jnp.pad(x, ((0, 0), (0, 0), (0, _D - x.shape[2])))
    table = (x.reshape(_PK, _PR, _D).transpose(1, 0, 2)
             .reshape(_PR, _PK * _D))

    self_idx = jnp.broadcast_to(
        jnp.arange(_S, dtype=NNsites.dtype)[:, None], (_S, _NP - _N))
    nn = jnp.concatenate([NNsites.T, self_idx], axis=1)
    zn = (jnp.arange(_B, dtype=jnp.int32)[:, None, None] * _S
          + nn[None, :, :]).reshape(_NIDX)
    idx = _PK * (zn % _PR) + zn // _PR

    ws, bs, nos = [], [], []
    for psi, bias in ((Psi0, b0), (Psi1, b1), (Psi2, b2), (Psi3, b3), (Psi4, b4)):
        o, cin, _ = psi.shape
        psip = jnp.pad(psi, ((0, 0), (0, _C - cin), (0, 0)))
        psig = psip[:, :, GnnPerms]
        w = jnp.transpose(psig, (3, 1, 2, 0)).reshape(_N * _C, _G * o)
        w = jnp.pad(w, ((0, (_NP - _N) * _C), (0, 0)))
        ws.append(w.astype(f32))
        bs.append(jnp.tile(bias, _G)[None, :].astype(f32))
        nos.append(o)

    sc_gather = _make_sc_gather()
    for l in range(5):
        rows = sc_gather(table.reshape(_NROWS, _D), idx).reshape(_NROWS, _NP * _D)
        table = _layer_tc(rows, ws[l], bs[l], nos[l])

    rows = sc_gather(table.reshape(_NROWS, _D), idx).reshape(_NROWS, _NP * _D)
    psirg = PsiR[:, GnnPerms]
    k1 = jnp.zeros((_N, _C, _K * _G), f32)
    k1 = k1.at[:, 0, :].set(jnp.transpose(psirg, (2, 0, 1)).reshape(_N, _K * _G))
    k1 = k1.reshape(_N * _C, _K * _G)
    k1 = jnp.pad(k1, ((0, (_NP - _N) * _C), (0, 0)))
    d1 = (jnp.einsum('kd,gde->kge', VR, gdiags) / _G).reshape(_K * _G, 3)
    shells2d = jnp.tile(SitesToShells.astype(jnp.int32), _B)[:, None]
    return _final_tc(rows, shells2d, k1, d1.astype(f32))
